# baseline (speedup 1.0000x reference)
import functools

import jax
import jax.numpy as jnp
from jax import lax
from jax.experimental import pallas as pl
from jax.experimental.pallas import tpu as pltpu
from jax.experimental.pallas import tpu_sc as plsc

_B, _S, _D, _L = 4, 8192, 128, 16
_C = 128
_DEPTH = 4


@functools.cache
def _make_kernel():
    info = plsc.get_sparse_core_info()
    nc, ns = info.num_cores, info.num_subcores
    nw = nc * ns
    p_per_w = _S // nw
    n_items = _B * p_per_w // _C
    n_halves = p_per_w // _C
    mesh = plsc.VectorSubcoreMesh(core_axis_name="c", subcore_axis_name="s")

    @functools.partial(
        pl.kernel,
        mesh=mesh,
        out_type=jax.ShapeDtypeStruct((_B, _S, _D), jnp.float32),
        scratch_types=[
            pltpu.VMEM((_B, p_per_w), jnp.int32),
            pltpu.VMEM((p_per_w, _D), jnp.float32),
            pltpu.VMEM((_DEPTH, _C, _D), jnp.float32),
            pltpu.SemaphoreType.DMA,
            pltpu.SemaphoreType.DMA((n_halves,)),
            pltpu.SemaphoreType.DMA((_DEPTH,)),
            pltpu.SemaphoreType.DMA((_DEPTH,)),
        ],
    )
    def k(idx_hbm, wemb_hbm, pemb_hbm, out_hbm,
          idx_v, pos_v, rows_v, isem, psem, gsem, osem):
        wid = lax.axis_index("s") * nc + lax.axis_index("c")
        pbase = wid * p_per_w

        def coords(j):
            return lax.rem(j, _B), j // _B

        def pos_desc(h):
            return pltpu.make_async_copy(
                pemb_hbm.at[pl.ds(pbase + h * _C, _C)],
                pos_v.at[pl.ds(h * _C, _C)], psem.at[h])

        def gather_desc(j):
            b, h = coords(j)
            buf = lax.rem(j, _DEPTH)
            return pltpu.make_async_copy(
                wemb_hbm.at[idx_v.at[b, pl.ds(h * _C, _C)]],
                rows_v.at[buf], gsem.at[buf])

        def out_desc(j):
            b, h = coords(j)
            buf = lax.rem(j, _DEPTH)
            return pltpu.make_async_copy(
                rows_v.at[buf], out_hbm.at[b, pl.ds(pbase + h * _C, _C)],
                osem.at[buf])

        for h in range(n_halves):
            pos_desc(h).start()
        pltpu.sync_copy(idx_hbm.at[:, pl.ds(pbase, p_per_w)], idx_v)

        def prime(j, c):
            gather_desc(j).start()
            return c
        lax.fori_loop(0, _DEPTH, prime, 0)

        def item(j, c):
            b, h = coords(j)
            buf = lax.rem(j, _DEPTH)

            @pl.when(lax.rem(j, _B) == 0)
            def _():
                pos_desc(h).wait()

            gather_desc(j).wait()

            @plsc.parallel_loop(0, _C, unroll=4)
            def add_body(r):
                prow = h * _C + r
                for col in range(_D // _L):
                    sl = pl.ds(col * _L, _L)
                    if col % 2 == 0:
                        plsc.addupdate(rows_v.at[buf, r, sl], pos_v[prow, sl])
                    else:
                        rows_v[buf, r, sl] = rows_v[buf, r, sl] + pos_v[prow, sl]

            out_desc(j).start()
            nxt = j + 2

            @pl.when(jnp.logical_and(nxt >= _DEPTH, nxt < n_items))
            def _():
                out_desc(nxt - _DEPTH).wait()
                gather_desc(nxt).start()

            return c
        lax.fori_loop(0, n_items, item, 0)

        def drain(j, c):
            out_desc(j).wait()
            return c
        lax.fori_loop(n_items - _DEPTH, n_items, drain, 0)

    return k


def kernel(input_ids, word_embeddings, position_embeddings):
    if input_ids.dtype != jnp.int32:
        input_ids = input_ids.astype(jnp.int32)
    return _make_kernel()(input_ids, word_embeddings, position_embeddings)


# speedup vs baseline: 1.0054x; 1.0054x over previous
import functools

import jax
import jax.numpy as jnp
from jax import lax
from jax.experimental import pallas as pl
from jax.experimental.pallas import tpu as pltpu
from jax.experimental.pallas import tpu_sc as plsc

_B, _S, _D, _L = 4, 8192, 128, 16
_C = 128
_DEPTH = 4


@functools.cache
def _make_kernel():
    info = plsc.get_sparse_core_info()
    nc, ns = info.num_cores, info.num_subcores
    nw = nc * ns
    p_per_w = _S // nw
    n_items = _B * p_per_w // _C
    n_halves = p_per_w // _C
    mesh = plsc.VectorSubcoreMesh(core_axis_name="c", subcore_axis_name="s")

    @functools.partial(
        pl.kernel,
        mesh=mesh,
        out_type=jax.ShapeDtypeStruct((_B, _S, _D), jnp.float32),
        scratch_types=[
            pltpu.VMEM((_B, p_per_w), jnp.int32),
            pltpu.VMEM((p_per_w, _D), jnp.float32),
            pltpu.VMEM((_DEPTH, _C, _D), jnp.float32),
            pltpu.SemaphoreType.DMA,
            pltpu.SemaphoreType.DMA((n_halves,)),
            pltpu.SemaphoreType.DMA((_DEPTH,)),
            pltpu.SemaphoreType.DMA((_DEPTH,)),
        ],
    )
    def k(idx_hbm, wemb_hbm, pemb_hbm, out_hbm,
          idx_v, pos_v, rows_v, isem, psem, gsem, osem):
        wid = lax.axis_index("s") * nc + lax.axis_index("c")
        pbase = wid * p_per_w

        def coords(j):
            return lax.rem(j, _B), j // _B

        def pos_desc(h):
            return pltpu.make_async_copy(
                pemb_hbm.at[pl.ds(pbase + h * _C, _C)],
                pos_v.at[pl.ds(h * _C, _C)], psem.at[h])

        def gather_desc(j):
            b, h = coords(j)
            buf = lax.rem(j, _DEPTH)
            return pltpu.make_async_copy(
                wemb_hbm.at[idx_v.at[b, pl.ds(h * _C, _C)]],
                rows_v.at[buf], gsem.at[buf])

        def out_desc(j):
            b, h = coords(j)
            buf = lax.rem(j, _DEPTH)
            return pltpu.make_async_copy(
                rows_v.at[buf], out_hbm.at[b, pl.ds(pbase + h * _C, _C)],
                osem.at[buf])

        for h in range(n_halves):
            pos_desc(h).start()
        pltpu.sync_copy(idx_hbm.at[:, pl.ds(pbase, p_per_w)], idx_v)

        def prime(j, c):
            gather_desc(j).start()
            return c
        lax.fori_loop(0, _DEPTH, prime, 0)

        def item(j, c):
            b, h = coords(j)
            buf = lax.rem(j, _DEPTH)

            @pl.when(lax.rem(j, _B) == 0)
            def _():
                pos_desc(h).wait()

            with jax.named_scope("gw"):
                gather_desc(j).wait()

            sc = jax.named_scope("add"); sc.__enter__()
            @plsc.parallel_loop(0, _C, unroll=4)
            def add_body(r):
                prow = h * _C + r
                for col in range(_D // _L):
                    sl = pl.ds(col * _L, _L)
                    plsc.addupdate(rows_v.at[buf, r, sl], pos_v[prow, sl])

            sc.__exit__(None, None, None)
            with jax.named_scope("ostart"):
                out_desc(j).start()
            nxt = j + 2

            with jax.named_scope("la"):
                @pl.when(jnp.logical_and(nxt >= _DEPTH, nxt < n_items))
                def _():
                    out_desc(nxt - _DEPTH).wait()
                    gather_desc(nxt).start()

            return c
        lax.fori_loop(0, n_items, item, 0)

        jax.named_scope("drain").__enter__()
        def drain(j, c):
            out_desc(j).wait()
            return c
        lax.fori_loop(n_items - _DEPTH, n_items, drain, 0)

    return k


def kernel(input_ids, word_embeddings, position_embeddings):
    if input_ids.dtype != jnp.int32:
        input_ids = input_ids.astype(jnp.int32)
    return _make_kernel()(input_ids, word_embeddings, position_embeddings)


# speedup vs baseline: 1.0350x; 1.0295x over previous
import functools

import jax
import jax.numpy as jnp
from jax import lax
from jax.experimental import pallas as pl
from jax.experimental.pallas import tpu as pltpu
from jax.experimental.pallas import tpu_sc as plsc

_B, _S, _D, _L = 4, 8192, 128, 16
_C = 128
_DEPTH = 5
_AHEAD = 3


@functools.cache
def _make_kernel():
    info = plsc.get_sparse_core_info()
    nc, ns = info.num_cores, info.num_subcores
    nw = nc * ns
    p_per_w = _S // nw
    n_items = _B * p_per_w // _C
    n_halves = p_per_w // _C
    mesh = plsc.VectorSubcoreMesh(core_axis_name="c", subcore_axis_name="s")

    @functools.partial(
        pl.kernel,
        mesh=mesh,
        out_type=jax.ShapeDtypeStruct((_B, _S, _D), jnp.float32),
        scratch_types=[
            pltpu.VMEM((_B, p_per_w), jnp.int32),
            pltpu.VMEM((p_per_w, _D), jnp.float32),
            pltpu.VMEM((_DEPTH, _C, _D), jnp.float32),
            pltpu.SemaphoreType.DMA,
            pltpu.SemaphoreType.DMA((n_halves,)),
            pltpu.SemaphoreType.DMA((_DEPTH,)),
            pltpu.SemaphoreType.DMA((_DEPTH,)),
        ],
    )
    def k(idx_hbm, wemb_hbm, pemb_hbm, out_hbm,
          idx_v, pos_v, rows_v, isem, psem, gsem, osem):
        wid = lax.axis_index("s") * nc + lax.axis_index("c")
        pbase = wid * p_per_w

        def coords(j):
            return lax.rem(j, _B), j // _B

        def pos_desc(h):
            return pltpu.make_async_copy(
                pemb_hbm.at[pl.ds(pbase + h * _C, _C)],
                pos_v.at[pl.ds(h * _C, _C)], psem.at[h])

        def gather_desc(j):
            b, h = coords(j)
            buf = lax.rem(j, _DEPTH)
            return pltpu.make_async_copy(
                wemb_hbm.at[idx_v.at[b, pl.ds(h * _C, _C)]],
                rows_v.at[buf], gsem.at[buf])

        def out_desc(j):
            b, h = coords(j)
            buf = lax.rem(j, _DEPTH)
            return pltpu.make_async_copy(
                rows_v.at[buf], out_hbm.at[b, pl.ds(pbase + h * _C, _C)],
                osem.at[buf])

        icopy = pltpu.make_async_copy(
            idx_hbm.at[:, pl.ds(pbase, p_per_w)], idx_v, isem)
        icopy.start()
        for h in range(n_halves):
            pos_desc(h).start()
        icopy.wait()

        def prime(j, c):
            gather_desc(j).start()
            return c
        lax.fori_loop(0, _DEPTH, prime, 0)

        def item(j, c):
            b, h = coords(j)
            buf = lax.rem(j, _DEPTH)

            @pl.when(lax.rem(j, _B) == 0)
            def _():
                pos_desc(h).wait()

            gather_desc(j).wait()

            @plsc.parallel_loop(0, _C, unroll=4)
            def add_body(r):
                prow = h * _C + r
                for col in range(_D // _L):
                    sl = pl.ds(col * _L, _L)
                    plsc.addupdate(rows_v.at[buf, r, sl], pos_v[prow, sl])

            out_desc(j).start()
            nxt = j + _AHEAD

            @pl.when(jnp.logical_and(nxt >= _DEPTH, nxt < n_items))
            def _():
                out_desc(nxt - _DEPTH).wait()
                gather_desc(nxt).start()

            return c
        lax.fori_loop(0, n_items, item, 0)

        def drain(j, c):
            out_desc(j).wait()
            return c
        lax.fori_loop(n_items - _DEPTH, n_items, drain, 0)

    return k


def kernel(input_ids, word_embeddings, position_embeddings):
    if input_ids.dtype != jnp.int32:
        input_ids = input_ids.astype(jnp.int32)
    return _make_kernel()(input_ids, word_embeddings, position_embeddings)


# speedup vs baseline: 1.0644x; 1.0284x over previous
import functools

import jax
import jax.numpy as jnp
from jax import lax
from jax.experimental import pallas as pl
from jax.experimental.pallas import tpu as pltpu
from jax.experimental.pallas import tpu_sc as plsc

_B, _S, _D, _L = 4, 8192, 128, 16
_C = 64
_DEPTH = 8
_AHEAD = 4


@functools.cache
def _make_kernel():
    info = plsc.get_sparse_core_info()
    nc, ns = info.num_cores, info.num_subcores
    nw = nc * ns
    p_per_w = _S // nw
    n_items = _B * p_per_w // _C
    n_halves = p_per_w // _C
    mesh = plsc.VectorSubcoreMesh(core_axis_name="c", subcore_axis_name="s")

    @functools.partial(
        pl.kernel,
        mesh=mesh,
        out_type=jax.ShapeDtypeStruct((_B, _S, _D), jnp.float32),
        scratch_types=[
            pltpu.VMEM((_B, p_per_w), jnp.int32),
            pltpu.VMEM((p_per_w, _D), jnp.float32),
            pltpu.VMEM((_DEPTH, _C, _D), jnp.float32),
            pltpu.SemaphoreType.DMA,
            pltpu.SemaphoreType.DMA((n_halves,)),
            pltpu.SemaphoreType.DMA((_DEPTH,)),
            pltpu.SemaphoreType.DMA((_DEPTH,)),
        ],
    )
    def k(idx_hbm, wemb_hbm, pemb_hbm, out_hbm,
          idx_v, pos_v, rows_v, isem, psem, gsem, osem):
        wid = lax.axis_index("s") * nc + lax.axis_index("c")
        pbase = wid * p_per_w

        def coords(j):
            return lax.rem(j, _B), j // _B

        def pos_desc(h):
            return pltpu.make_async_copy(
                pemb_hbm.at[pl.ds(pbase + h * _C, _C)],
                pos_v.at[pl.ds(h * _C, _C)], psem.at[h])

        def gather_desc(j):
            b, h = coords(j)
            buf = lax.rem(j, _DEPTH)
            return pltpu.make_async_copy(
                wemb_hbm.at[idx_v.at[b, pl.ds(h * _C, _C)]],
                rows_v.at[buf], gsem.at[buf])

        def out_desc(j):
            b, h = coords(j)
            buf = lax.rem(j, _DEPTH)
            return pltpu.make_async_copy(
                rows_v.at[buf], out_hbm.at[b, pl.ds(pbase + h * _C, _C)],
                osem.at[buf])

        icopy = pltpu.make_async_copy(
            idx_hbm.at[:, pl.ds(pbase, p_per_w)], idx_v, isem)
        icopy.start()
        for h in range(n_halves):
            pos_desc(h).start()
        icopy.wait()

        def prime(j, c):
            gather_desc(j).start()
            return c
        lax.fori_loop(0, _DEPTH, prime, 0)

        def item(j, c):
            b, h = coords(j)
            buf = lax.rem(j, _DEPTH)

            @pl.when(lax.rem(j, _B) == 0)
            def _():
                pos_desc(h).wait()

            gather_desc(j).wait()

            @plsc.parallel_loop(0, _C, unroll=4)
            def add_body(r):
                prow = h * _C + r
                for col in range(_D // _L):
                    sl = pl.ds(col * _L, _L)
                    plsc.addupdate(rows_v.at[buf, r, sl], pos_v[prow, sl])

            out_desc(j).start()
            nxt = j + _AHEAD

            @pl.when(jnp.logical_and(nxt >= _DEPTH, nxt < n_items))
            def _():
                out_desc(nxt - _DEPTH).wait()
                gather_desc(nxt).start()

            return c
        lax.fori_loop(0, n_items, item, 0)

        def drain(j, c):
            out_desc(j).wait()
            return c
        lax.fori_loop(n_items - _DEPTH, n_items, drain, 0)

    return k


def kernel(input_ids, word_embeddings, position_embeddings):
    if input_ids.dtype != jnp.int32:
        input_ids = input_ids.astype(jnp.int32)
    return _make_kernel()(input_ids, word_embeddings, position_embeddings)
